# Initial kernel scaffold; baseline (speedup 1.0000x reference)
import jax
import jax.numpy as jnp
from jax.experimental import pallas as pl


def kernel(data):
    raise NotImplementedError("write your pallas kernel here")


# speedup vs baseline: 1405.7446x; 1405.7446x over previous
import jax
import jax.numpy as jnp
from jax.experimental import pallas as pl
from jax.experimental.pallas import tpu as pltpu

ROWS, COLS = 2048, 8192
BLOCK_ROWS = 256
GRID = ROWS // BLOCK_ROWS
NEDGES = 129


def _edges():
    return jnp.concatenate(
        [-jnp.logspace(6, -7, 64), jnp.array([0.0]), jnp.logspace(-7, 6, 64)]
    )


def _hist_body(edges_ref, data_ref, counts_ref, stats_ref, acc_ref):
    pi = pl.program_id(0)

    @pl.when(pi == 0)
    def _init():
        for i in range(NEDGES + 1):
            acc_ref[i] = 0.0
        stats_ref[0] = jnp.inf
        stats_ref[1] = -jnp.inf
        stats_ref[2] = 0.0
        stats_ref[3] = 0.0

    block = data_ref[...]
    for i in range(NEDGES):
        acc_ref[i] += jnp.sum((block >= edges_ref[i]).astype(jnp.float32))
    acc_ref[NEDGES] += jnp.sum((block > edges_ref[NEDGES - 1]).astype(jnp.float32))

    stats_ref[0] = jnp.minimum(stats_ref[0], jnp.min(block))
    stats_ref[1] = jnp.maximum(stats_ref[1], jnp.max(block))
    stats_ref[2] += jnp.sum(block)
    stats_ref[3] += jnp.sum(block * block)

    @pl.when(pi == GRID - 1)
    def _fin():
        for i in range(NEDGES - 2):
            counts_ref[i] = acc_ref[i] - acc_ref[i + 1]
        counts_ref[NEDGES - 2] = acc_ref[NEDGES - 2] - acc_ref[NEDGES]


def kernel(data):
    edges = _edges()
    counts, stats = pl.pallas_call(
        _hist_body,
        grid=(GRID,),
        in_specs=[
            pl.BlockSpec(memory_space=pltpu.SMEM),
            pl.BlockSpec((BLOCK_ROWS, COLS), lambda i: (i, 0)),
        ],
        out_specs=[
            pl.BlockSpec(memory_space=pltpu.SMEM),
            pl.BlockSpec(memory_space=pltpu.SMEM),
        ],
        out_shape=[
            jax.ShapeDtypeStruct((NEDGES - 1,), jnp.float32),
            jax.ShapeDtypeStruct((4,), jnp.float32),
        ],
        scratch_shapes=[pltpu.SMEM((NEDGES + 1,), jnp.float32)],
    )(edges, data)
    num = jnp.asarray(data.size, jnp.int32)
    return (stats[0], stats[1], num, stats[2], stats[3], edges, counts)
